# Initial kernel scaffold; baseline (speedup 1.0000x reference)
import functools

import jax
import jax.numpy as jnp
from jax import lax
from jax.experimental import pallas as pl
from jax.experimental.pallas import tpu as pltpu
from jax.experimental.pallas import tpu_sc as plsc


def _sc_spmm(table, src, dst, ew, n_nodes, n_ch):
    info = plsc.get_sparse_core_info()
    NC, NS = info.num_cores, info.num_subcores
    NW = NC * NS
    E = src.shape[0]
    EPW = E // NW
    B = 80
    NB = EPW // B
    RPS = n_nodes // NS
    CH = 125
    NCH = RPS // CH
    CZ = n_ch // 16

    mesh = plsc.VectorSubcoreMesh(core_axis_name="c", subcore_axis_name="s")

    @functools.partial(
        pl.kernel,
        mesh=mesh,
        out_type=jax.ShapeDtypeStruct((NC * n_nodes, n_ch), jnp.float32),
        scratch_types=[
            pltpu.VMEM((B,), jnp.int32),
            pltpu.VMEM((B,), jnp.int32),
            pltpu.VMEM((B,), jnp.float32),
            pltpu.VMEM((B, n_ch), jnp.float32),
            pltpu.VMEM((CH, n_ch), jnp.float32),
            pltpu.VMEM_SHARED((n_nodes, n_ch), jnp.float32),
            pltpu.SemaphoreType.DMA,
        ],
    )
    def spmm_kernel(tab_hbm, src_hbm, dst_hbm, ew_hbm, out_hbm,
                    sidx, didx, wv, rows, bounce, acc, sem):
        c = lax.axis_index("c")
        s = lax.axis_index("s")
        wid = s * NC + c

        @pl.loop(0, CH)
        def _zero_bounce(i):
            for j in range(CZ):
                bounce[i, pl.ds(16 * j, 16)] = jnp.zeros((16,), jnp.float32)

        @pl.loop(0, NCH)
        def _zero_acc(k):
            pltpu.sync_copy(bounce, acc.at[pl.ds(s * RPS + k * CH, CH)])

        plsc.subcore_barrier()

        @pl.loop(0, NB)
        def _edges(i):
            base = wid * EPW + i * B
            pltpu.sync_copy(src_hbm.at[pl.ds(base, B)], sidx)
            pltpu.sync_copy(dst_hbm.at[pl.ds(base, B)], didx)
            pltpu.sync_copy(ew_hbm.at[pl.ds(base, B)], wv)
            pltpu.async_copy(tab_hbm.at[sidx], rows, sem).wait()

            @pl.loop(0, B)
            def _scale(r):
                w = jnp.full((16,), wv[r], jnp.float32)
                for j in range(CZ):
                    rows[r, pl.ds(16 * j, 16)] = rows[r, pl.ds(16 * j, 16)] * w

            pltpu.sync_copy(rows, acc.at[didx], add=True)

        plsc.subcore_barrier()

        @pl.loop(0, NCH)
        def _flush(k):
            r0 = s * RPS + k * CH
            pltpu.sync_copy(acc.at[pl.ds(r0, CH)], bounce)
            pltpu.sync_copy(bounce, out_hbm.at[pl.ds(c * n_nodes + r0, CH)])

    out = spmm_kernel(table, src, dst, ew)
    return out.reshape(NC, n_nodes, n_ch)


def _mm_body(x_ref, w_ref, o_ref):
    o_ref[...] = jnp.dot(x_ref[...], w_ref[...],
                         preferred_element_type=jnp.float32)


def _tc_matmul(x, w, blk):
    n, kdim = x.shape
    m = w.shape[1]
    grid = n // blk
    return pl.pallas_call(
        _mm_body,
        grid=(grid,),
        in_specs=[
            pl.BlockSpec((blk, kdim), lambda i: (i, 0)),
            pl.BlockSpec((kdim, m), lambda i: (0, 0)),
        ],
        out_specs=pl.BlockSpec((blk, m), lambda i: (i, 0)),
        out_shape=jax.ShapeDtypeStruct((n, m), jnp.float32),
    )(x, w)


def _merge_body(p_ref, b_ref, w_ref, o_ref):
    h = jnp.maximum(p_ref[0] + p_ref[1] + b_ref[...], 0.0)
    o_ref[...] = jnp.dot(h, w_ref[...], preferred_element_type=jnp.float32)


def _tc_merge_relu_mm(partials, b1, w2b, blk):
    _, n, kdim = partials.shape
    m = w2b.shape[1]
    grid = n // blk
    return pl.pallas_call(
        _merge_body,
        grid=(grid,),
        in_specs=[
            pl.BlockSpec((2, blk, kdim), lambda i: (0, i, 0)),
            pl.BlockSpec((1, kdim), lambda i: (0, 0)),
            pl.BlockSpec((kdim, m), lambda i: (0, 0)),
        ],
        out_specs=pl.BlockSpec((blk, m), lambda i: (i, 0)),
        out_shape=jax.ShapeDtypeStruct((n, m), jnp.float32),
    )(partials, b1, w2b)


def _lsm_body(p_ref, o_ref):
    s = p_ref[0] + p_ref[1]
    m = jnp.max(s)
    e = jnp.exp(s - m)
    t = jnp.sum(e) * (1.0 / 16.0)
    o_ref[...] = s - (m + jnp.log(t))


def _tc_log_softmax(partials):
    _, n, m = partials.shape
    return pl.pallas_call(
        _lsm_body,
        out_shape=jax.ShapeDtypeStruct((n, m), jnp.float32),
    )(partials)


def kernel(features, edge_index, edge_weight, W1, b1, W2, b2):
    n = features.shape[0]
    src = edge_index[0].astype(jnp.int32)
    dst = edge_index[1].astype(jnp.int32)
    ew = edge_weight.astype(jnp.float32)

    support = _tc_matmul(features, W1, blk=1000)

    p1 = _sc_spmm(support, src, dst, ew, n, support.shape[1])

    w2b = jnp.tile(W2, (1, 16))
    y16 = _tc_merge_relu_mm(p1, b1.reshape(1, -1), w2b, blk=1000)

    p2 = _sc_spmm(y16, src, dst, ew, n, 16)

    out16 = _tc_log_softmax(p2)
    return out16[:, :1]


# speedup vs baseline: 6.2229x; 6.2229x over previous
import functools

import jax
import jax.numpy as jnp
from jax import lax
from jax.experimental import pallas as pl
from jax.experimental.pallas import tpu as pltpu
from jax.experimental.pallas import tpu_sc as plsc


def _sc_spmm(table, src, dst, ew, n_nodes, n_ch):
    info = plsc.get_sparse_core_info()
    NC, NS = info.num_cores, info.num_subcores
    NW = NC * NS
    E = src.shape[0]
    EPW = E // NW
    B = 80
    NB = EPW // B
    RPS = n_nodes // NS
    CH = 125
    NCH = RPS // CH
    CZ = n_ch // 16

    mesh = plsc.VectorSubcoreMesh(core_axis_name="c", subcore_axis_name="s")

    @functools.partial(
        pl.kernel,
        mesh=mesh,
        compiler_params=pltpu.CompilerParams(use_tc_tiling_on_sc=False),
        out_type=jax.ShapeDtypeStruct((NC * n_nodes, n_ch), jnp.float32),
        scratch_types=[
            pltpu.VMEM((B,), jnp.int32),
            pltpu.VMEM((B,), jnp.int32),
            pltpu.VMEM((B,), jnp.float32),
            pltpu.VMEM((B, n_ch), jnp.float32),
            pltpu.VMEM((CH, n_ch), jnp.float32),
            pltpu.VMEM_SHARED((n_nodes, n_ch), jnp.float32),
            pltpu.SemaphoreType.DMA,
        ],
    )
    def spmm_kernel(tab_hbm, src_hbm, dst_hbm, ew_hbm, out_hbm,
                    sidx, didx, wv, rows, bounce, acc, sem):
        c = lax.axis_index("c")
        s = lax.axis_index("s")
        wid = s * NC + c

        @pl.loop(0, CH)
        def _zero_bounce(i):
            for j in range(CZ):
                bounce[i, pl.ds(16 * j, 16)] = jnp.zeros((16,), jnp.float32)

        @pl.loop(0, NCH)
        def _zero_acc(k):
            pltpu.sync_copy(bounce, acc.at[pl.ds(s * RPS + k * CH, CH)])

        plsc.subcore_barrier()

        @pl.loop(0, NB)
        def _edges(i):
            base = wid * EPW + i * B
            pltpu.sync_copy(src_hbm.at[pl.ds(base, B)], sidx)
            pltpu.sync_copy(dst_hbm.at[pl.ds(base, B)], didx)
            pltpu.sync_copy(ew_hbm.at[pl.ds(base, B)], wv)
            pltpu.async_copy(tab_hbm.at[sidx], rows, sem).wait()

            @pl.loop(0, B // 16)
            def _scale(g):
                wch = wv[pl.ds(16 * g, 16)]
                for r in range(16):
                    w = jnp.full((16,), wch[r], jnp.float32)
                    row = 16 * g + r
                    for j in range(CZ):
                        rows[row, pl.ds(16 * j, 16)] = (
                            rows[row, pl.ds(16 * j, 16)] * w)

            pltpu.sync_copy(rows, acc.at[didx], add=True)

        plsc.subcore_barrier()

        @pl.loop(0, NCH)
        def _flush(k):
            r0 = s * RPS + k * CH
            pltpu.sync_copy(acc.at[pl.ds(r0, CH)], bounce)
            pltpu.sync_copy(bounce, out_hbm.at[pl.ds(c * n_nodes + r0, CH)])

    out = spmm_kernel(table, src, dst, ew)
    return out.reshape(NC, n_nodes, n_ch)


def _mm_body(x_ref, w_ref, o_ref):
    o_ref[...] = jnp.dot(x_ref[...], w_ref[...],
                         preferred_element_type=jnp.float32)


def _tc_matmul(x, w, blk):
    n, kdim = x.shape
    m = w.shape[1]
    grid = n // blk
    return pl.pallas_call(
        _mm_body,
        grid=(grid,),
        in_specs=[
            pl.BlockSpec((blk, kdim), lambda i: (i, 0)),
            pl.BlockSpec((kdim, m), lambda i: (0, 0)),
        ],
        out_specs=pl.BlockSpec((blk, m), lambda i: (i, 0)),
        out_shape=jax.ShapeDtypeStruct((n, m), jnp.float32),
    )(x, w)


def _merge_body(p_ref, b_ref, w_ref, o_ref):
    h = jnp.maximum(p_ref[0] + p_ref[1] + b_ref[...], 0.0)
    o_ref[...] = jnp.dot(h, w_ref[...], preferred_element_type=jnp.float32)


def _tc_merge_relu_mm(partials, b1, w2b, blk):
    _, n, kdim = partials.shape
    m = w2b.shape[1]
    grid = n // blk
    return pl.pallas_call(
        _merge_body,
        grid=(grid,),
        in_specs=[
            pl.BlockSpec((2, blk, kdim), lambda i: (0, i, 0)),
            pl.BlockSpec((1, kdim), lambda i: (0, 0)),
            pl.BlockSpec((kdim, m), lambda i: (0, 0)),
        ],
        out_specs=pl.BlockSpec((blk, m), lambda i: (i, 0)),
        out_shape=jax.ShapeDtypeStruct((n, m), jnp.float32),
    )(partials, b1, w2b)


def _lsm_body(p_ref, o_ref):
    s = p_ref[0] + p_ref[1]
    m = jnp.max(s)
    e = jnp.exp(s - m)
    t = jnp.sum(e) * (1.0 / 16.0)
    o_ref[...] = s - (m + jnp.log(t))


def _tc_log_softmax(partials):
    _, n, m = partials.shape
    return pl.pallas_call(
        _lsm_body,
        out_shape=jax.ShapeDtypeStruct((n, m), jnp.float32),
    )(partials)


def kernel(features, edge_index, edge_weight, W1, b1, W2, b2):
    n = features.shape[0]
    src = edge_index[0].astype(jnp.int32)
    dst = edge_index[1].astype(jnp.int32)
    ew = edge_weight.astype(jnp.float32)

    support = _tc_matmul(features, W1, blk=1000)

    p1 = _sc_spmm(support, src, dst, ew, n, support.shape[1])

    w2b = jnp.tile(W2, (1, 16))
    y16 = _tc_merge_relu_mm(p1, b1.reshape(1, -1), w2b, blk=1000)

    p2 = _sc_spmm(y16, src, dst, ew, n, 16)

    out16 = _tc_log_softmax(p2)
    return out16[:, :1]


# speedup vs baseline: 12.1107x; 1.9462x over previous
import functools

import jax
import jax.numpy as jnp
from jax import lax
from jax.experimental import pallas as pl
from jax.experimental.pallas import tpu as pltpu
from jax.experimental.pallas import tpu_sc as plsc


def _sc_spmm(table, src, dst, ew, n_nodes, n_ch):
    info = plsc.get_sparse_core_info()
    NC, NS = info.num_cores, info.num_subcores
    NW = NC * NS
    E = src.shape[0]
    EPW = E // NW
    B = 80
    NB = EPW // B
    assert NB % 2 == 1
    RPS = n_nodes // NS
    CZ = n_ch // 16

    mesh = plsc.VectorSubcoreMesh(core_axis_name="c", subcore_axis_name="s")

    @functools.partial(
        pl.kernel,
        mesh=mesh,
        compiler_params=pltpu.CompilerParams(use_tc_tiling_on_sc=False),
        out_type=jax.ShapeDtypeStruct((NC * n_nodes, n_ch), jnp.float32),
        scratch_types=[
            pltpu.VMEM((NB, B), jnp.int32),
            pltpu.VMEM((NB, B), jnp.int32),
            pltpu.VMEM((NB, B), jnp.float32),
            pltpu.VMEM((B, n_ch), jnp.float32),
            pltpu.VMEM((B, n_ch), jnp.float32),
            pltpu.VMEM_SHARED((n_nodes, n_ch), jnp.float32),
            pltpu.SemaphoreType.DMA,
            pltpu.SemaphoreType.DMA,
            pltpu.SemaphoreType.DMA,
            pltpu.SemaphoreType.DMA,
        ],
    )
    def spmm_kernel(tab_hbm, src_hbm, dst_hbm, ew_hbm, zeros_hbm, out_hbm,
                    sidx, didx, wv, rowsA, rowsB, acc,
                    gsemA, gsemB, ssemA, ssemB):
        c = lax.axis_index("c")
        s = lax.axis_index("s")
        wid = s * NC + c

        pltpu.sync_copy(src_hbm.at[wid], sidx)
        pltpu.sync_copy(dst_hbm.at[wid], didx)
        pltpu.sync_copy(ew_hbm.at[wid], wv)

        pltpu.sync_copy(zeros_hbm, acc.at[pl.ds(s * RPS, RPS)])

        plsc.subcore_barrier()

        def scale(i, buf):
            @pl.loop(0, B // 16)
            def _scale(g):
                wch = wv[i, pl.ds(16 * g, 16)]
                for r in range(16):
                    w = jnp.full((16,), wch[r], jnp.float32)
                    row = 16 * g + r
                    for j in range(CZ):
                        buf[row, pl.ds(16 * j, 16)] = (
                            buf[row, pl.ds(16 * j, 16)] * w)

        def wait_gather(buf, gsem):
            pltpu.make_async_copy(tab_hbm.at[pl.ds(0, B)], buf, gsem).wait()

        def wait_scatter(buf, ssem):
            pltpu.make_async_copy(buf, acc.at[pl.ds(0, B)], ssem).wait()

        def step(i, bufX, gsemX, ssemX, bufY, gsemY, ssemY):
            wait_gather(bufX, gsemX)
            scale(i, bufX)
            pltpu.async_copy(bufX, acc.at[didx.at[i]], ssemX, add=True)

            @pl.when(i > 0)
            def _():
                wait_scatter(bufY, ssemY)

            @pl.when(i + 1 < NB)
            def _():
                pltpu.async_copy(tab_hbm.at[sidx.at[i + 1]], bufY, gsemY)

        pltpu.async_copy(tab_hbm.at[sidx.at[0]], rowsA, gsemA)

        @pl.loop(0, (NB - 1) // 2)
        def _edges(p):
            i0 = 2 * p
            step(i0, rowsA, gsemA, ssemA, rowsB, gsemB, ssemB)
            step(i0 + 1, rowsB, gsemB, ssemB, rowsA, gsemA, ssemA)

        wait_gather(rowsA, gsemA)
        scale(NB - 1, rowsA)
        pltpu.sync_copy(rowsA, acc.at[didx.at[NB - 1]], add=True)
        wait_scatter(rowsB, ssemB)

        plsc.subcore_barrier()

        pltpu.sync_copy(acc.at[pl.ds(s * RPS, RPS)],
                        out_hbm.at[pl.ds(c * n_nodes + s * RPS, RPS)])

    out = spmm_kernel(table,
                      src.reshape(NW, NB, B),
                      dst.reshape(NW, NB, B),
                      ew.reshape(NW, NB, B),
                      jnp.zeros((RPS, n_ch), jnp.float32))
    return out.reshape(NC, n_nodes, n_ch)


def _mm_body(x_ref, w_ref, o_ref):
    o_ref[...] = jnp.dot(x_ref[...], w_ref[...],
                         preferred_element_type=jnp.float32)


def _tc_matmul(x, w, blk):
    n, kdim = x.shape
    m = w.shape[1]
    grid = n // blk
    return pl.pallas_call(
        _mm_body,
        grid=(grid,),
        in_specs=[
            pl.BlockSpec((blk, kdim), lambda i: (i, 0)),
            pl.BlockSpec((kdim, m), lambda i: (0, 0)),
        ],
        out_specs=pl.BlockSpec((blk, m), lambda i: (i, 0)),
        out_shape=jax.ShapeDtypeStruct((n, m), jnp.float32),
    )(x, w)


def _merge_body(p_ref, b_ref, w_ref, o_ref):
    h = jnp.maximum(p_ref[0] + p_ref[1] + b_ref[...], 0.0)
    o_ref[...] = jnp.dot(h, w_ref[...], preferred_element_type=jnp.float32)


def _tc_merge_relu_mm(partials, b1, w2b, blk):
    _, n, kdim = partials.shape
    m = w2b.shape[1]
    grid = n // blk
    return pl.pallas_call(
        _merge_body,
        grid=(grid,),
        in_specs=[
            pl.BlockSpec((2, blk, kdim), lambda i: (0, i, 0)),
            pl.BlockSpec((1, kdim), lambda i: (0, 0)),
            pl.BlockSpec((kdim, m), lambda i: (0, 0)),
        ],
        out_specs=pl.BlockSpec((blk, m), lambda i: (i, 0)),
        out_shape=jax.ShapeDtypeStruct((n, m), jnp.float32),
    )(partials, b1, w2b)


def _lsm_body(p_ref, o_ref):
    s = p_ref[0] + p_ref[1]
    m = jnp.max(s)
    e = jnp.exp(s - m)
    t = jnp.sum(e) * (1.0 / 16.0)
    o_ref[...] = s - (m + jnp.log(t))


def _tc_log_softmax(partials):
    _, n, m = partials.shape
    return pl.pallas_call(
        _lsm_body,
        out_shape=jax.ShapeDtypeStruct((n, m), jnp.float32),
    )(partials)


def kernel(features, edge_index, edge_weight, W1, b1, W2, b2):
    n = features.shape[0]
    src = edge_index[0].astype(jnp.int32)
    dst = edge_index[1].astype(jnp.int32)
    ew = edge_weight.astype(jnp.float32)

    support = _tc_matmul(features, W1, blk=1000)

    p1 = _sc_spmm(support, src, dst, ew, n, support.shape[1])

    w2b = jnp.tile(W2, (1, 16))
    y16 = _tc_merge_relu_mm(p1, b1.reshape(1, -1), w2b, blk=1000)

    p2 = _sc_spmm(y16, src, dst, ew, n, 16)

    out16 = _tc_log_softmax(p2)
    return out16[:, :1]


# speedup vs baseline: 12.7895x; 1.0560x over previous
import functools

import jax
import jax.numpy as jnp
from jax import lax
from jax.experimental import pallas as pl
from jax.experimental.pallas import tpu as pltpu
from jax.experimental.pallas import tpu_sc as plsc


def _sc_spmm(table, src, dst, ew, n_nodes, n_ch, B):
    info = plsc.get_sparse_core_info()
    NC, NS = info.num_cores, info.num_subcores
    NW = NC * NS
    E = src.shape[0]
    EPW = E // NW
    NB = EPW // B
    assert NB % 2 == 1
    RPS = n_nodes // NS
    CZ = n_ch // 16

    mesh = plsc.VectorSubcoreMesh(core_axis_name="c", subcore_axis_name="s")

    @functools.partial(
        pl.kernel,
        mesh=mesh,
        compiler_params=pltpu.CompilerParams(use_tc_tiling_on_sc=False),
        out_type=jax.ShapeDtypeStruct((NC * n_nodes, n_ch), jnp.float32),
        scratch_types=[
            pltpu.VMEM((NB, B), jnp.int32),
            pltpu.VMEM((NB, B), jnp.int32),
            pltpu.VMEM((NB, B), jnp.float32),
            pltpu.VMEM((B, n_ch), jnp.float32),
            pltpu.VMEM((B, n_ch), jnp.float32),
            pltpu.VMEM_SHARED((n_nodes, n_ch), jnp.float32),
            pltpu.SemaphoreType.DMA,
            pltpu.SemaphoreType.DMA,
            pltpu.SemaphoreType.DMA,
            pltpu.SemaphoreType.DMA,
        ],
    )
    def spmm_kernel(tab_hbm, src_hbm, dst_hbm, ew_hbm, zeros_hbm, out_hbm,
                    sidx, didx, wv, rowsA, rowsB, acc,
                    gsemA, gsemB, ssemA, ssemB):
        c = lax.axis_index("c")
        s = lax.axis_index("s")
        wid = s * NC + c

        pltpu.sync_copy(src_hbm.at[wid], sidx)
        pltpu.sync_copy(dst_hbm.at[wid], didx)
        pltpu.sync_copy(ew_hbm.at[wid], wv)

        pltpu.sync_copy(zeros_hbm, acc.at[pl.ds(s * RPS, RPS)])

        plsc.subcore_barrier()

        def scale(i, buf):
            @plsc.parallel_loop(0, B // 16, unroll=2)
            def _scale(g):
                wch = wv[i, pl.ds(16 * g, 16)]
                for r in range(16):
                    w = jnp.full((16,), wch[r], jnp.float32)
                    row = 16 * g + r
                    for j in range(CZ):
                        buf[row, pl.ds(16 * j, 16)] = (
                            buf[row, pl.ds(16 * j, 16)] * w)

        def wait_gather(buf, gsem):
            pltpu.make_async_copy(tab_hbm.at[pl.ds(0, B)], buf, gsem).wait()

        def wait_scatter(buf, ssem):
            pltpu.make_async_copy(buf, acc.at[pl.ds(0, B)], ssem).wait()

        def step(i, bufX, gsemX, ssemX, bufY, gsemY, ssemY):
            wait_gather(bufX, gsemX)
            scale(i, bufX)
            pltpu.async_copy(bufX, acc.at[didx.at[i]], ssemX, add=True)

            @pl.when(i > 0)
            def _():
                wait_scatter(bufY, ssemY)

            @pl.when(i + 1 < NB)
            def _():
                pltpu.async_copy(tab_hbm.at[sidx.at[i + 1]], bufY, gsemY)

        pltpu.async_copy(tab_hbm.at[sidx.at[0]], rowsA, gsemA)

        @pl.loop(0, (NB - 1) // 2)
        def _edges(p):
            i0 = 2 * p
            step(i0, rowsA, gsemA, ssemA, rowsB, gsemB, ssemB)
            step(i0 + 1, rowsB, gsemB, ssemB, rowsA, gsemA, ssemA)

        wait_gather(rowsA, gsemA)
        scale(NB - 1, rowsA)
        pltpu.sync_copy(rowsA, acc.at[didx.at[NB - 1]], add=True)
        wait_scatter(rowsB, ssemB)

        plsc.subcore_barrier()

        pltpu.sync_copy(acc.at[pl.ds(s * RPS, RPS)],
                        out_hbm.at[pl.ds(c * n_nodes + s * RPS, RPS)])

    out = spmm_kernel(table,
                      src.reshape(NW, NB, B),
                      dst.reshape(NW, NB, B),
                      ew.reshape(NW, NB, B),
                      jnp.zeros((RPS, n_ch), jnp.float32))
    return out.reshape(NC, n_nodes, n_ch)


def _mm_body(x_ref, w_ref, o_ref):
    o_ref[...] = jnp.dot(x_ref[...], w_ref[...],
                         preferred_element_type=jnp.float32)


def _tc_matmul(x, w, blk):
    n, kdim = x.shape
    m = w.shape[1]
    grid = n // blk
    return pl.pallas_call(
        _mm_body,
        grid=(grid,),
        in_specs=[
            pl.BlockSpec((blk, kdim), lambda i: (i, 0)),
            pl.BlockSpec((kdim, m), lambda i: (0, 0)),
        ],
        out_specs=pl.BlockSpec((blk, m), lambda i: (i, 0)),
        out_shape=jax.ShapeDtypeStruct((n, m), jnp.float32),
    )(x, w)


def _merge_body(p_ref, b_ref, w_ref, o_ref):
    h = jnp.maximum(p_ref[0] + p_ref[1] + b_ref[...], 0.0)
    o_ref[...] = jnp.dot(h, w_ref[...], preferred_element_type=jnp.float32)


def _tc_merge_relu_mm(partials, b1, w2b, blk):
    _, n, kdim = partials.shape
    m = w2b.shape[1]
    grid = n // blk
    return pl.pallas_call(
        _merge_body,
        grid=(grid,),
        in_specs=[
            pl.BlockSpec((2, blk, kdim), lambda i: (0, i, 0)),
            pl.BlockSpec((1, kdim), lambda i: (0, 0)),
            pl.BlockSpec((kdim, m), lambda i: (0, 0)),
        ],
        out_specs=pl.BlockSpec((blk, m), lambda i: (i, 0)),
        out_shape=jax.ShapeDtypeStruct((n, m), jnp.float32),
    )(partials, b1, w2b)


def _lsm_body(p_ref, o_ref):
    s = p_ref[0] + p_ref[1]
    m = jnp.max(s)
    e = jnp.exp(s - m)
    t = jnp.sum(e) * (1.0 / 16.0)
    o_ref[...] = s - (m + jnp.log(t))


def _tc_log_softmax(partials):
    _, n, m = partials.shape
    return pl.pallas_call(
        _lsm_body,
        out_shape=jax.ShapeDtypeStruct((n, m), jnp.float32),
    )(partials)


def kernel(features, edge_index, edge_weight, W1, b1, W2, b2):
    n = features.shape[0]
    src = edge_index[0].astype(jnp.int32)
    dst = edge_index[1].astype(jnp.int32)
    ew = edge_weight.astype(jnp.float32)

    support = _tc_matmul(features, W1, blk=1000)

    p1 = _sc_spmm(support, src, dst, ew, n, support.shape[1], B=80)

    w2b = jnp.tile(W2, (1, 16))
    y16 = _tc_merge_relu_mm(p1, b1.reshape(1, -1), w2b, blk=1000)

    e = src.shape[0]
    nb2 = -(-e // (32 * 128))
    if nb2 % 2 == 0:
        nb2 += 1
    pad = 32 * nb2 * 128 - e
    src2 = jnp.concatenate([src, jnp.zeros((pad,), jnp.int32)])
    dst2 = jnp.concatenate([dst, jnp.zeros((pad,), jnp.int32)])
    ew2 = jnp.concatenate([ew, jnp.zeros((pad,), jnp.float32)])
    p2 = _sc_spmm(y16, src2, dst2, ew2, n, 16, B=128)

    out16 = _tc_log_softmax(p2)
    return out16[:, :1]


# speedup vs baseline: 13.5401x; 1.0587x over previous
import functools

import jax
import jax.numpy as jnp
from jax import lax
from jax.experimental import pallas as pl
from jax.experimental.pallas import tpu as pltpu
from jax.experimental.pallas import tpu_sc as plsc


def _sc_spmm(table, pk, ew, n_nodes, n_ch, B, D, AH):
    info = plsc.get_sparse_core_info()
    NC, NS = info.num_cores, info.num_subcores
    NW = NC * NS
    NB = pk.shape[1]
    assert NB % D == 0 and 1 <= AH <= D - 1
    RPS = n_nodes // NS
    CZ = n_ch // 16

    mesh = plsc.VectorSubcoreMesh(core_axis_name="c", subcore_axis_name="s")

    @functools.partial(
        pl.kernel,
        mesh=mesh,
        compiler_params=pltpu.CompilerParams(use_tc_tiling_on_sc=False),
        out_type=jax.ShapeDtypeStruct((NC * n_nodes, n_ch), jnp.float32),
        scratch_types=[
            pltpu.VMEM((NB, B), jnp.int32),
            pltpu.VMEM((D, B, n_ch), jnp.float32),
            pltpu.VMEM((D, B), jnp.int32),
            pltpu.VMEM((D, B), jnp.int32),
            pltpu.VMEM((D, B), jnp.float32),
            pltpu.VMEM_SHARED((n_nodes, n_ch), jnp.float32),
            pltpu.SemaphoreType.DMA((D,)),
            pltpu.SemaphoreType.DMA((D,)),
        ],
    )
    def spmm_kernel(tab_hbm, pk_hbm, ew_hbm, zeros_hbm, out_hbm,
                    pks, rows, sidx, didx, wv, acc, gsem, ssem):
        c = lax.axis_index("c")
        s = lax.axis_index("s")
        wid = s * NC + c

        pltpu.sync_copy(pk_hbm.at[wid], pks)

        def issue(j, k):
            for g in range(B // 16):
                v = pks[j, pl.ds(16 * g, 16)]
                sidx[k, pl.ds(16 * g, 16)] = v & 0xFFFF
                didx[k, pl.ds(16 * g, 16)] = lax.shift_right_logical(v, 16)
            pltpu.async_copy(ew_hbm.at[wid, j], wv.at[k], gsem.at[k])
            pltpu.async_copy(tab_hbm.at[sidx.at[k]], rows.at[k], gsem.at[k])

        def wait_gather(k):
            pltpu.make_async_copy(tab_hbm.at[pl.ds(0, B)], rows.at[k],
                                  gsem.at[k]).wait()
            pltpu.make_async_copy(ew_hbm.at[0, 0], wv.at[k],
                                  gsem.at[k]).wait()

        def wait_scatter(k):
            pltpu.make_async_copy(rows.at[k], acc.at[pl.ds(0, B)],
                                  ssem.at[k]).wait()

        def scale(k, buf):
            @plsc.parallel_loop(0, B // 16, unroll=2)
            def _scale(g):
                wch = wv[k, pl.ds(16 * g, 16)]
                for r in range(16):
                    w = jnp.full((16,), wch[r], jnp.float32)
                    row = 16 * g + r
                    for j in range(CZ):
                        buf[row, pl.ds(16 * j, 16)] = (
                            buf[row, pl.ds(16 * j, 16)] * w)

        for j in range(AH):
            issue(j, j)

        pltpu.sync_copy(zeros_hbm, acc.at[pl.ds(s * RPS, RPS)])
        plsc.subcore_barrier()

        @pl.loop(0, NB // D)
        def _edges(t):
            i0 = t * D
            for k in range(D):
                i = i0 + k
                wait_gather(k)
                scale(k, rows.at[k])
                pltpu.async_copy(rows.at[k], acc.at[didx.at[k]],
                                 ssem.at[k], add=True)
                kf = (k + AH) % D

                @pl.when(i + AH < NB)
                def _():
                    @pl.when(i + AH >= D)
                    def _():
                        wait_scatter(kf)
                    issue(i + AH, kf)

        for k in range(D):
            wait_scatter(k)

        plsc.subcore_barrier()

        pltpu.sync_copy(acc.at[pl.ds(s * RPS, RPS)],
                        out_hbm.at[pl.ds(c * n_nodes + s * RPS, RPS)])

    out = spmm_kernel(table, pk, ew, jnp.zeros((RPS, n_ch), jnp.float32))
    return out.reshape(NC, n_nodes, n_ch)


def _pack_edges(src, dst, ew, B, D):
    NW = 32
    e = src.shape[0]
    nb = -(-e // (NW * B))
    nb = ((nb + D - 1) // D) * D
    pad = NW * nb * B - e
    pk = jnp.bitwise_or(src, jnp.left_shift(dst, 16))
    pk = jnp.concatenate([pk, jnp.zeros((pad,), jnp.int32)])
    ewp = jnp.concatenate([ew, jnp.zeros((pad,), jnp.float32)])
    return pk.reshape(NW, nb, B), ewp.reshape(NW, nb, B)


def _mm_body(x_ref, w_ref, o_ref):
    o_ref[...] = jnp.dot(x_ref[...], w_ref[...],
                         preferred_element_type=jnp.float32)


def _tc_matmul(x, w, blk):
    n, kdim = x.shape
    m = w.shape[1]
    grid = n // blk
    return pl.pallas_call(
        _mm_body,
        grid=(grid,),
        in_specs=[
            pl.BlockSpec((blk, kdim), lambda i: (i, 0)),
            pl.BlockSpec((kdim, m), lambda i: (0, 0)),
        ],
        out_specs=pl.BlockSpec((blk, m), lambda i: (i, 0)),
        out_shape=jax.ShapeDtypeStruct((n, m), jnp.float32),
    )(x, w)


def _merge_body(p_ref, b_ref, w_ref, o_ref):
    h = jnp.maximum(p_ref[0] + p_ref[1] + b_ref[...], 0.0)
    o_ref[...] = jnp.dot(h, w_ref[...], preferred_element_type=jnp.float32)


def _tc_merge_relu_mm(partials, b1, w2b, blk):
    _, n, kdim = partials.shape
    m = w2b.shape[1]
    grid = n // blk
    return pl.pallas_call(
        _merge_body,
        grid=(grid,),
        in_specs=[
            pl.BlockSpec((2, blk, kdim), lambda i: (0, i, 0)),
            pl.BlockSpec((1, kdim), lambda i: (0, 0)),
            pl.BlockSpec((kdim, m), lambda i: (0, 0)),
        ],
        out_specs=pl.BlockSpec((blk, m), lambda i: (i, 0)),
        out_shape=jax.ShapeDtypeStruct((n, m), jnp.float32),
    )(partials, b1, w2b)


def _lsm_body(p_ref, o_ref):
    s = p_ref[0] + p_ref[1]
    m = jnp.max(s)
    e = jnp.exp(s - m)
    t = jnp.sum(e) * (1.0 / 16.0)
    o_ref[...] = s - (m + jnp.log(t))


def _tc_log_softmax(partials):
    _, n, m = partials.shape
    return pl.pallas_call(
        _lsm_body,
        out_shape=jax.ShapeDtypeStruct((n, m), jnp.float32),
    )(partials)


def kernel(features, edge_index, edge_weight, W1, b1, W2, b2):
    n = features.shape[0]
    src = edge_index[0].astype(jnp.int32)
    dst = edge_index[1].astype(jnp.int32)
    ew = edge_weight.astype(jnp.float32)

    support = _tc_matmul(features, W1, blk=1000)

    pk1, ew1 = _pack_edges(src, dst, ew, B=80, D=3)
    p1 = _sc_spmm(support, pk1, ew1, n, support.shape[1], B=80, D=3, AH=2)

    w2b = jnp.tile(W2, (1, 16))
    y16 = _tc_merge_relu_mm(p1, b1.reshape(1, -1), w2b, blk=1000)

    pk2, ew2 = _pack_edges(src, dst, ew, B=128, D=8)
    p2 = _sc_spmm(y16, pk2, ew2, n, 16, B=128, D=8, AH=6)

    out16 = _tc_log_softmax(p2)
    return out16[:, :1]
